# baseline (speedup 1.0000x reference)
import jax
import jax.numpy as jnp
from jax.experimental import pallas as pl
from jax.experimental.pallas import tpu as pltpu

B, S, D, H = 16, 512, 256, 256
TS = 256


def _lstm_kernel(x_ref, m_ref, wih_ref, whh_ref, b_ref,
                 out_ref, hN_ref, cN_ref,
                 h_ref, c_ref, xpre_ref, outs_ref):
    @pl.when(pl.program_id(0) == 0)
    def _init():
        h_ref[...] = jnp.zeros_like(h_ref)
        c_ref[...] = jnp.zeros_like(c_ref)

    wih = wih_ref[...]
    bias = b_ref[...]
    for g in range(0, TS, 8):
        xt = jnp.swapaxes(x_ref[:, g:g + 8, :], 0, 1).reshape(8 * B, D)
        xp = jnp.dot(xt, wih, preferred_element_type=jnp.float32)
        xpre_ref[g:g + 8] = xp.reshape(8, B, 4 * H) + bias

    whh = whh_ref[...]

    def step(t, carry):
        h, c = carry
        gates = xpre_ref[t] + jnp.dot(h, whh,
                                      preferred_element_type=jnp.float32)
        i = jax.nn.sigmoid(gates[:, 0:H])
        f = jax.nn.sigmoid(gates[:, H:2 * H])
        g = jnp.tanh(gates[:, 2 * H:3 * H])
        o = jax.nn.sigmoid(gates[:, 3 * H:4 * H])
        c_new = f * c + i * g
        h_new = o * jnp.tanh(c_new)
        m2 = m_ref[t]
        outs_ref[t] = h_new * m2
        h = m2 * h_new + (1.0 - m2) * h
        c = m2 * c_new + (1.0 - m2) * c
        return h, c

    h, c = jax.lax.fori_loop(0, TS, step, (h_ref[...], c_ref[...]),
                             unroll=16)
    h_ref[...] = h
    c_ref[...] = c
    hN_ref[...] = h
    cN_ref[...] = c
    out_ref[...] = jnp.swapaxes(outs_ref[...], 0, 1)


def kernel(inputs, mask, W_ih, W_hh, b_ih, b_hh):
    m_tm = jnp.swapaxes(mask, 0, 1).astype(inputs.dtype)[..., None]
    wih_t = W_ih.T
    whh_t = W_hh.T
    b = (b_ih + b_hh)[None, None, :]

    grid = (S // TS,)
    out, hN, cN = pl.pallas_call(
        _lstm_kernel,
        grid=grid,
        in_specs=[
            pl.BlockSpec((B, TS, D), lambda i: (0, i, 0)),
            pl.BlockSpec((TS, B, 1), lambda i: (i, 0, 0)),
            pl.BlockSpec((D, 4 * H), lambda i: (0, 0)),
            pl.BlockSpec((H, 4 * H), lambda i: (0, 0)),
            pl.BlockSpec((1, 1, 4 * H), lambda i: (0, 0, 0)),
        ],
        out_specs=[
            pl.BlockSpec((B, TS, H), lambda i: (0, i, 0)),
            pl.BlockSpec((B, H), lambda i: (0, 0)),
            pl.BlockSpec((B, H), lambda i: (0, 0)),
        ],
        out_shape=[
            jax.ShapeDtypeStruct((B, S, H), jnp.float32),
            jax.ShapeDtypeStruct((B, H), jnp.float32),
            jax.ShapeDtypeStruct((B, H), jnp.float32),
        ],
        scratch_shapes=[
            pltpu.VMEM((B, H), jnp.float32),
            pltpu.VMEM((B, H), jnp.float32),
            pltpu.VMEM((TS, B, 4 * H), jnp.float32),
            pltpu.VMEM((TS, B, H), jnp.float32),
        ],
    )(inputs, m_tm, wih_t, whh_t, b)

    return out, hN[None, :, :], cN[None, :, :]


# speedup vs baseline: 1.0107x; 1.0107x over previous
import jax
import jax.numpy as jnp
from jax.experimental import pallas as pl
from jax.experimental.pallas import tpu as pltpu

B, S, D, H = 16, 512, 256, 256
TS = 128


def _lstm_kernel(x_ref, m_ref, wih_ref, whh_ref, b_ref,
                 out_ref, hN_ref, cN_ref,
                 h_ref, c_ref, xpre_ref, outs_ref):
    @pl.when(pl.program_id(0) == 0)
    def _init():
        h_ref[...] = jnp.zeros_like(h_ref)
        c_ref[...] = jnp.zeros_like(c_ref)

    wih = wih_ref[...]
    bias = b_ref[...]
    for g in range(0, TS, 8):
        xt = jnp.swapaxes(x_ref[:, g:g + 8, :], 0, 1).reshape(8 * B, D)
        xp = jnp.dot(xt, wih, preferred_element_type=jnp.float32)
        xpre_ref[g:g + 8] = xp.reshape(8, B, 4 * H) + bias

    whh = whh_ref[...]

    def step(t, carry):
        h, c = carry
        gates = xpre_ref[t] + jnp.dot(h, whh,
                                      preferred_element_type=jnp.float32)
        i = jax.nn.sigmoid(gates[:, 0:H])
        f = jax.nn.sigmoid(gates[:, H:2 * H])
        g = jnp.tanh(gates[:, 2 * H:3 * H])
        o = jax.nn.sigmoid(gates[:, 3 * H:4 * H])
        c_new = f * c + i * g
        h_new = o * jnp.tanh(c_new)
        m2 = m_ref[t]
        out_t = h_new * m2
        outs_ref[t] = out_t
        km = 1.0 - m2
        h = out_t + km * h
        c = m2 * c_new + km * c
        return h, c

    h, c = jax.lax.fori_loop(0, TS, step, (h_ref[...], c_ref[...]),
                             unroll=16)
    h_ref[...] = h
    c_ref[...] = c
    hN_ref[...] = h
    cN_ref[...] = c
    out_ref[...] = jnp.swapaxes(outs_ref[...], 0, 1)


def kernel(inputs, mask, W_ih, W_hh, b_ih, b_hh):
    m_tm = jnp.swapaxes(mask, 0, 1).astype(inputs.dtype)[..., None]
    wih_t = W_ih.T
    whh_t = W_hh.T
    b = (b_ih + b_hh)[None, None, :]

    grid = (S // TS,)
    out, hN, cN = pl.pallas_call(
        _lstm_kernel,
        grid=grid,
        in_specs=[
            pl.BlockSpec((B, TS, D), lambda i: (0, i, 0)),
            pl.BlockSpec((TS, B, 1), lambda i: (i, 0, 0)),
            pl.BlockSpec((D, 4 * H), lambda i: (0, 0)),
            pl.BlockSpec((H, 4 * H), lambda i: (0, 0)),
            pl.BlockSpec((1, 1, 4 * H), lambda i: (0, 0, 0)),
        ],
        out_specs=[
            pl.BlockSpec((B, TS, H), lambda i: (0, i, 0)),
            pl.BlockSpec((B, H), lambda i: (0, 0)),
            pl.BlockSpec((B, H), lambda i: (0, 0)),
        ],
        out_shape=[
            jax.ShapeDtypeStruct((B, S, H), jnp.float32),
            jax.ShapeDtypeStruct((B, H), jnp.float32),
            jax.ShapeDtypeStruct((B, H), jnp.float32),
        ],
        scratch_shapes=[
            pltpu.VMEM((B, H), jnp.float32),
            pltpu.VMEM((B, H), jnp.float32),
            pltpu.VMEM((TS, B, 4 * H), jnp.float32),
            pltpu.VMEM((TS, B, H), jnp.float32),
        ],
    )(inputs, m_tm, wih_t, whh_t, b)

    return out, hN[None, :, :], cN[None, :, :]


# speedup vs baseline: 1.0164x; 1.0057x over previous
import jax
import jax.numpy as jnp
from jax.experimental import pallas as pl
from jax.experimental.pallas import tpu as pltpu

B, S, D, H = 16, 512, 256, 256
TS = 128


def _lstm_kernel(x_ref, m_ref, wih_ref, whh_ref, b_ref,
                 out_ref, hN_ref, cN_ref,
                 h_ref, c_ref, xpre_ref, outs_ref):
    @pl.when(pl.program_id(0) == 0)
    def _init():
        h_ref[...] = jnp.zeros_like(h_ref)
        c_ref[...] = jnp.zeros_like(c_ref)

    wih = wih_ref[...]
    bias = b_ref[...]
    for g in range(0, TS, 8):
        xt = jnp.swapaxes(x_ref[:, g:g + 8, :], 0, 1).reshape(8 * B, D)
        xp = jnp.dot(xt, wih, preferred_element_type=jnp.float32)
        xpre_ref[g:g + 8] = xp.reshape(8, B, 4 * H) + bias

    whh = whh_ref[...]

    def step(t, carry):
        h, c = carry
        gates = xpre_ref[t] + jnp.dot(h, whh,
                                      preferred_element_type=jnp.float32)
        i = jax.nn.sigmoid(gates[:, 0:H])
        f = jax.nn.sigmoid(gates[:, H:2 * H])
        g = jnp.tanh(gates[:, 2 * H:3 * H])
        o = jax.nn.sigmoid(gates[:, 3 * H:4 * H])
        c_new = f * c + i * g
        h_new = o * jnp.tanh(c_new)
        m2 = m_ref[t]
        out_t = h_new * m2
        outs_ref[t] = out_t
        km = 1.0 - m2
        h = out_t + km * h
        c = m2 * c_new + km * c
        return h, c

    h, c = jax.lax.fori_loop(0, TS, step, (h_ref[...], c_ref[...]),
                             unroll=32)
    h_ref[...] = h
    c_ref[...] = c
    hN_ref[...] = h
    cN_ref[...] = c
    out_ref[...] = jnp.swapaxes(outs_ref[...], 0, 1)


def kernel(inputs, mask, W_ih, W_hh, b_ih, b_hh):
    m_tm = jnp.swapaxes(mask, 0, 1).astype(inputs.dtype)[..., None]
    wih_t = W_ih.T
    whh_t = W_hh.T
    b = (b_ih + b_hh)[None, None, :]

    grid = (S // TS,)
    out, hN, cN = pl.pallas_call(
        _lstm_kernel,
        grid=grid,
        in_specs=[
            pl.BlockSpec((B, TS, D), lambda i: (0, i, 0)),
            pl.BlockSpec((TS, B, 1), lambda i: (i, 0, 0)),
            pl.BlockSpec((D, 4 * H), lambda i: (0, 0)),
            pl.BlockSpec((H, 4 * H), lambda i: (0, 0)),
            pl.BlockSpec((1, 1, 4 * H), lambda i: (0, 0, 0)),
        ],
        out_specs=[
            pl.BlockSpec((B, TS, H), lambda i: (0, i, 0)),
            pl.BlockSpec((B, H), lambda i: (0, 0)),
            pl.BlockSpec((B, H), lambda i: (0, 0)),
        ],
        out_shape=[
            jax.ShapeDtypeStruct((B, S, H), jnp.float32),
            jax.ShapeDtypeStruct((B, H), jnp.float32),
            jax.ShapeDtypeStruct((B, H), jnp.float32),
        ],
        scratch_shapes=[
            pltpu.VMEM((B, H), jnp.float32),
            pltpu.VMEM((B, H), jnp.float32),
            pltpu.VMEM((TS, B, 4 * H), jnp.float32),
            pltpu.VMEM((TS, B, H), jnp.float32),
        ],
    )(inputs, m_tm, wih_t, whh_t, b)

    return out, hN[None, :, :], cN[None, :, :]
